# baseline (speedup 1.0000x reference)
import functools

import jax
import jax.numpy as jnp
from jax import lax
from jax.experimental import pallas as pl
from jax.experimental.pallas import tpu as pltpu
from jax.experimental.pallas import tpu_sc as plsc

N_NODES = 10000
N_EDGES = 320000
D = 128

NUM_CORES = 2
NUM_SUBCORES = 16
NUM_WORKERS = NUM_CORES * NUM_SUBCORES

SLOTS = 4
CHUNK = 64
CHUNKS_PER_WORKER = 160
N_EDGES_PAD = NUM_WORKERS * CHUNKS_PER_WORKER * CHUNK
N_ROWS_AGG = N_NODES + 16
ROWS_MAIN = 624
ROWS_TAIL = N_NODES - NUM_SUBCORES * ROWS_MAIN

_SC_MESH = plsc.VectorSubcoreMesh(core_axis_name="c", subcore_axis_name="s")


@functools.partial(
    pl.kernel,
    mesh=_SC_MESH,
    out_type=jax.ShapeDtypeStruct((NUM_CORES * N_NODES, D), jnp.float32),
    scratch_types=(
        [pltpu.VMEM((CHUNK, D), jnp.float32) for _ in range(SLOTS)]
        + [pltpu.VMEM((CHUNK,), jnp.int32) for _ in range(SLOTS)]
        + [pltpu.VMEM((CHUNK,), jnp.int32) for _ in range(SLOTS)]
        + [pltpu.VMEM_SHARED((N_ROWS_AGG, D), jnp.float32)]
        + [pltpu.SemaphoreType.DMA for _ in range(4 * SLOTS)]
    ),
)
def _sc_aggregate(x_hbm, src_hbm, dst_hbm, zeros_hbm, out_hbm,
                  r0, r1, r2, r3, s0, s1, s2, s3, d0, d1, d2, d3, agg_sh,
                  gs0, gs1, gs2, gs3, cs0, cs1, cs2, cs3,
                  is0, is1, is2, is3, ds0, ds1, ds2, ds3):
    rows = (r0, r1, r2, r3)
    srcb = (s0, s1, s2, s3)
    dstb = (d0, d1, d2, d3)
    gsem = (gs0, gs1, gs2, gs3)
    ssem = (cs0, cs1, cs2, cs3)
    sisem = (is0, is1, is2, is3)
    sdsem = (ds0, ds1, ds2, ds3)

    cid = lax.axis_index("c")
    sid = lax.axis_index("s")
    wid = sid * NUM_CORES + cid
    base_c = wid * CHUNKS_PER_WORKER

    def start_src(k, t):
        off = (base_c + t) * CHUNK
        pltpu.async_copy(src_hbm.at[pl.ds(off, CHUNK)], srcb[k], sisem[k])

    def wait_src(k):
        pltpu.make_async_copy(src_hbm.at[pl.ds(0, CHUNK)], srcb[k],
                              sisem[k]).wait()

    def start_dst(k, t):
        off = (base_c + t) * CHUNK
        pltpu.async_copy(dst_hbm.at[pl.ds(off, CHUNK)], dstb[k], sdsem[k])

    def wait_dst(k):
        pltpu.make_async_copy(dst_hbm.at[pl.ds(0, CHUNK)], dstb[k],
                              sdsem[k]).wait()

    def start_gather(k):
        pltpu.async_copy(x_hbm.at[srcb[k]], rows[k], gsem[k])

    def wait_gather(k):
        pltpu.make_async_copy(x_hbm.at[pl.ds(0, CHUNK)], rows[k],
                              gsem[k]).wait()

    def start_scatter(k):
        pltpu.async_copy(rows[k], agg_sh.at[dstb[k]], ssem[k], add=True)

    def wait_scatter(k):
        pltpu.make_async_copy(x_hbm.at[pl.ds(0, CHUNK)], rows[k],
                              ssem[k]).wait()

    row0 = sid * ROWS_MAIN
    pltpu.sync_copy(zeros_hbm, agg_sh.at[pl.ds(row0, ROWS_MAIN)])

    @pl.when(sid == NUM_SUBCORES - 1)
    def _zero_tail():
        pltpu.sync_copy(zeros_hbm.at[pl.ds(0, ROWS_TAIL)],
                        agg_sh.at[pl.ds(NUM_SUBCORES * ROWS_MAIN, ROWS_TAIL)])

    for k in range(SLOTS):
        start_src(k, k)
    start_dst(0, 0)
    start_dst(1, 1)
    plsc.subcore_barrier()
    wait_src(0)
    start_gather(0)
    wait_src(1)
    start_gather(1)

    def body(i, carry):
        for k in range(SLOTS):
            t = SLOTS * i + k
            k2 = (k + 2) % SLOTS
            wait_gather(k)
            start_src(k, lax.rem(t + 4, CHUNKS_PER_WORKER))
            if k < 2:
                @pl.when(i > 0)
                def _():
                    wait_scatter(k2)
            else:
                wait_scatter(k2)
            start_dst(k2, lax.rem(t + 2, CHUNKS_PER_WORKER))
            wait_dst(k)
            start_scatter(k)
            wait_src(k2)
            start_gather(k2)
        return carry

    lax.fori_loop(0, CHUNKS_PER_WORKER // SLOTS, body, 0)

    wait_scatter(2)
    wait_scatter(3)
    wait_gather(0)
    wait_gather(1)
    wait_src(2)
    wait_src(3)
    wait_dst(0)
    wait_dst(1)

    plsc.subcore_barrier()
    pltpu.sync_copy(agg_sh.at[pl.ds(row0, ROWS_MAIN)],
                    out_hbm.at[pl.ds(cid * N_NODES + row0, ROWS_MAIN)])

    @pl.when(sid == NUM_SUBCORES - 1)
    def _copy_tail():
        pltpu.sync_copy(
            agg_sh.at[pl.ds(NUM_SUBCORES * ROWS_MAIN, ROWS_TAIL)],
            out_hbm.at[pl.ds(cid * N_NODES + NUM_SUBCORES * ROWS_MAIN,
                             ROWS_TAIL)])


def _mlp_body(eps_ref, x_ref, a0_ref, a1_ref, w1_ref, b1_ref, w2_ref, b2_ref,
              o_ref):
    h = x_ref[...] * eps_ref[0] + a0_ref[...] + a1_ref[...]
    h = jnp.dot(h, w1_ref[...], preferred_element_type=jnp.float32) + b1_ref[...]
    h = jnp.where(h > 0, h, 0.01 * h)
    h = jnp.dot(h, w2_ref[...], preferred_element_type=jnp.float32) + b2_ref[...]
    h = jnp.where(h > 0, h, 0.01 * h)
    o_ref[...] = h


_BLK = 1000

_mlp_call = pl.pallas_call(
    _mlp_body,
    out_shape=jax.ShapeDtypeStruct((N_NODES, D), jnp.float32),
    grid=(N_NODES // _BLK,),
    in_specs=[
        pl.BlockSpec(memory_space=pltpu.SMEM),
        pl.BlockSpec((_BLK, D), lambda i: (i, 0)),
        pl.BlockSpec((_BLK, D), lambda i: (i, 0)),
        pl.BlockSpec((_BLK, D), lambda i: (i, 0)),
        pl.BlockSpec((D, D), lambda i: (0, 0)),
        pl.BlockSpec((1, D), lambda i: (0, 0)),
        pl.BlockSpec((D, D), lambda i: (0, 0)),
        pl.BlockSpec((1, D), lambda i: (0, 0)),
    ],
    out_specs=pl.BlockSpec((_BLK, D), lambda i: (i, 0)),
)


def kernel(x, edge_index, eps, W1, b1, W2, b2):
    npad = N_EDGES_PAD - N_EDGES
    src = jnp.concatenate(
        [edge_index[0], jnp.zeros((npad,), jnp.int32)])
    dst = jnp.concatenate(
        [edge_index[1], jnp.full((npad,), N_NODES, jnp.int32)])
    zeros = jnp.zeros((ROWS_MAIN, D), jnp.float32)
    agg = _sc_aggregate(x, src, dst, zeros)
    scale = jnp.reshape(1.0 + eps, (1,)).astype(jnp.float32)
    out = _mlp_call(scale, x, agg[:N_NODES], agg[N_NODES:],
                    W1.T, b1.reshape(1, D), W2.T, b2.reshape(1, D))
    return out


# speedup vs baseline: 1.9684x; 1.9684x over previous
import functools

import jax
import jax.numpy as jnp
from jax import lax
from jax.experimental import pallas as pl
from jax.experimental.pallas import tpu as pltpu
from jax.experimental.pallas import tpu_sc as plsc

N_NODES = 10000
N_EDGES = 320000
D = 128

NUM_CORES = 2
NUM_SUBCORES = 16
NUM_WORKERS = NUM_CORES * NUM_SUBCORES

CHUNK = 104
EDGES_PER_WORKER = N_EDGES // NUM_WORKERS
CHUNKS_PER_WORKER = 97
EDGES_PER_WORKER_PAD = CHUNKS_PER_WORKER * CHUNK
PAIRS = (CHUNKS_PER_WORKER - 1) // 2
N_ROWS_AGG = N_NODES + 16
ROWS_MAIN = 624
ROWS_TAIL = N_NODES - NUM_SUBCORES * ROWS_MAIN

_SC_MESH = plsc.VectorSubcoreMesh(core_axis_name="c", subcore_axis_name="s")


@functools.partial(
    pl.kernel,
    mesh=_SC_MESH,
    out_type=jax.ShapeDtypeStruct((NUM_CORES * N_NODES, D), jnp.float32),
    scratch_types=[
        pltpu.VMEM((CHUNKS_PER_WORKER, CHUNK), jnp.int32),
        pltpu.VMEM((CHUNK,), jnp.int32),
        pltpu.VMEM((CHUNK,), jnp.int32),
        pltpu.VMEM((CHUNK, D), jnp.float32),
        pltpu.VMEM((CHUNK, D), jnp.float32),
        pltpu.VMEM_SHARED((N_ROWS_AGG, D), jnp.float32),
        pltpu.SemaphoreType.DMA,
        pltpu.SemaphoreType.DMA,
        pltpu.SemaphoreType.DMA,
        pltpu.SemaphoreType.DMA,
    ],
)
def _sc_aggregate(x_hbm, src_hbm, dst_hbm, zeros_hbm, out_hbm,
                  src_v, dst_a, dst_b, rows_a, rows_b, agg_sh,
                  sem_a, sem_b, sem_da, sem_db):
    cid = lax.axis_index("c")
    sid = lax.axis_index("s")
    wid = sid * NUM_CORES + cid

    row0 = sid * ROWS_MAIN
    pltpu.sync_copy(zeros_hbm, agg_sh.at[pl.ds(row0, ROWS_MAIN)])

    @pl.when(sid == NUM_SUBCORES - 1)
    def _zero_tail():
        pltpu.sync_copy(zeros_hbm.at[pl.ds(0, ROWS_TAIL)],
                        agg_sh.at[pl.ds(NUM_SUBCORES * ROWS_MAIN, ROWS_TAIL)])

    pltpu.sync_copy(src_hbm.at[wid], src_v)
    plsc.subcore_barrier()

    def wait_gather(rows_v, sem):
        pltpu.make_async_copy(x_hbm.at[pl.ds(0, CHUNK)], rows_v, sem).wait()

    def start_dst(j, dst_v, sem):
        off = wid * EDGES_PER_WORKER_PAD + j * CHUNK
        pltpu.async_copy(dst_hbm.at[pl.ds(off, CHUNK)], dst_v, sem)

    def wait_dst(dst_v, sem):
        pltpu.make_async_copy(dst_hbm.at[pl.ds(0, CHUNK)], dst_v, sem).wait()

    start_dst(0, dst_a, sem_da)
    pltpu.async_copy(x_hbm.at[src_v.at[0]], rows_a, sem_a)

    def body(i, carry):
        j0 = 2 * i
        start_dst(j0 + 1, dst_b, sem_db)
        pltpu.async_copy(x_hbm.at[src_v.at[j0 + 1]], rows_b, sem_b)
        wait_gather(rows_a, sem_a)
        wait_dst(dst_a, sem_da)
        pltpu.sync_copy(rows_a, agg_sh.at[dst_a], add=True)
        start_dst(j0 + 2, dst_a, sem_da)
        pltpu.async_copy(x_hbm.at[src_v.at[j0 + 2]], rows_a, sem_a)
        wait_gather(rows_b, sem_b)
        wait_dst(dst_b, sem_db)
        pltpu.sync_copy(rows_b, agg_sh.at[dst_b], add=True)
        return carry

    lax.fori_loop(0, PAIRS, body, 0)
    wait_gather(rows_a, sem_a)
    wait_dst(dst_a, sem_da)
    pltpu.sync_copy(rows_a, agg_sh.at[dst_a], add=True)

    plsc.subcore_barrier()
    pltpu.sync_copy(agg_sh.at[pl.ds(row0, ROWS_MAIN)],
                    out_hbm.at[pl.ds(cid * N_NODES + row0, ROWS_MAIN)])

    @pl.when(sid == NUM_SUBCORES - 1)
    def _copy_tail():
        pltpu.sync_copy(
            agg_sh.at[pl.ds(NUM_SUBCORES * ROWS_MAIN, ROWS_TAIL)],
            out_hbm.at[pl.ds(cid * N_NODES + NUM_SUBCORES * ROWS_MAIN,
                             ROWS_TAIL)])


def _mlp_body(eps_ref, x_ref, a0_ref, a1_ref, w1_ref, b1_ref, w2_ref, b2_ref,
              o_ref):
    h = x_ref[...] * eps_ref[0] + a0_ref[...] + a1_ref[...]
    h = jnp.dot(h, w1_ref[...], preferred_element_type=jnp.float32) + b1_ref[...]
    h = jnp.where(h > 0, h, 0.01 * h)
    h = jnp.dot(h, w2_ref[...], preferred_element_type=jnp.float32) + b2_ref[...]
    h = jnp.where(h > 0, h, 0.01 * h)
    o_ref[...] = h


_BLK = 1000

_mlp_call = pl.pallas_call(
    _mlp_body,
    out_shape=jax.ShapeDtypeStruct((N_NODES, D), jnp.float32),
    grid=(N_NODES // _BLK,),
    in_specs=[
        pl.BlockSpec(memory_space=pltpu.SMEM),
        pl.BlockSpec((_BLK, D), lambda i: (i, 0)),
        pl.BlockSpec((_BLK, D), lambda i: (i, 0)),
        pl.BlockSpec((_BLK, D), lambda i: (i, 0)),
        pl.BlockSpec((D, D), lambda i: (0, 0)),
        pl.BlockSpec((1, D), lambda i: (0, 0)),
        pl.BlockSpec((D, D), lambda i: (0, 0)),
        pl.BlockSpec((1, D), lambda i: (0, 0)),
    ],
    out_specs=pl.BlockSpec((_BLK, D), lambda i: (i, 0)),
)


def kernel(x, edge_index, eps, W1, b1, W2, b2):
    npad = EDGES_PER_WORKER_PAD - EDGES_PER_WORKER
    src = jnp.concatenate(
        [edge_index[0].reshape(NUM_WORKERS, EDGES_PER_WORKER),
         jnp.zeros((NUM_WORKERS, npad), jnp.int32)], axis=1,
    ).reshape(NUM_WORKERS, CHUNKS_PER_WORKER, CHUNK)
    dst = jnp.concatenate(
        [edge_index[1].reshape(NUM_WORKERS, EDGES_PER_WORKER),
         jnp.full((NUM_WORKERS, npad), N_NODES, jnp.int32)], axis=1,
    ).reshape(-1)
    zeros = jnp.zeros((ROWS_MAIN, D), jnp.float32)
    agg = _sc_aggregate(x, src, dst, zeros)
    scale = jnp.reshape(1.0 + eps, (1,)).astype(jnp.float32)
    out = _mlp_call(scale, x, agg[:N_NODES], agg[N_NODES:],
                    W1.T, b1.reshape(1, D), W2.T, b2.reshape(1, D))
    return out


# speedup vs baseline: 3.0575x; 1.5533x over previous
import functools

import jax
import jax.numpy as jnp
from jax import lax
from jax.experimental import pallas as pl
from jax.experimental.pallas import tpu as pltpu
from jax.experimental.pallas import tpu_sc as plsc

N_NODES = 10000
N_EDGES = 320000
D = 128

NUM_CORES = 2
NUM_SUBCORES = 16
NUM_WORKERS = NUM_CORES * NUM_SUBCORES

CHUNK = 80
EDGES_PER_WORKER = N_EDGES // NUM_WORKERS
CHUNKS_PER_WORKER = EDGES_PER_WORKER // CHUNK
PAIRS = (CHUNKS_PER_WORKER - 1) // 2
ROWS_MAIN = 624
ROWS_TAIL = N_NODES - NUM_SUBCORES * ROWS_MAIN

_SC_MESH = plsc.VectorSubcoreMesh(core_axis_name="c", subcore_axis_name="s")


@functools.partial(
    pl.kernel,
    mesh=_SC_MESH,
    out_type=jax.ShapeDtypeStruct((NUM_CORES * N_NODES, D), jnp.float32),
    scratch_types=[
        pltpu.VMEM((CHUNKS_PER_WORKER, CHUNK), jnp.int32),
        pltpu.VMEM((CHUNK,), jnp.int32),
        pltpu.VMEM((CHUNK,), jnp.int32),
        pltpu.VMEM((CHUNK, D), jnp.float32),
        pltpu.VMEM((CHUNK, D), jnp.float32),
        pltpu.VMEM_SHARED((N_NODES, D), jnp.float32),
        pltpu.SemaphoreType.DMA,
        pltpu.SemaphoreType.DMA,
        pltpu.SemaphoreType.DMA,
        pltpu.SemaphoreType.DMA,
    ],
)
def _sc_aggregate(x_hbm, src_hbm, dst_hbm, zeros_hbm, out_hbm,
                  src_v, dst_a, dst_b, rows_a, rows_b, agg_sh,
                  sem_a, sem_b, sem_da, sem_db):
    cid = lax.axis_index("c")
    sid = lax.axis_index("s")
    wid = sid * NUM_CORES + cid

    row0 = sid * ROWS_MAIN
    pltpu.sync_copy(zeros_hbm, agg_sh.at[pl.ds(row0, ROWS_MAIN)])

    @pl.when(sid == NUM_SUBCORES - 1)
    def _zero_tail():
        pltpu.sync_copy(zeros_hbm.at[pl.ds(0, ROWS_TAIL)],
                        agg_sh.at[pl.ds(NUM_SUBCORES * ROWS_MAIN, ROWS_TAIL)])

    pltpu.sync_copy(src_hbm.at[wid], src_v)
    plsc.subcore_barrier()

    def wait_gather(rows_v, sem):
        pltpu.make_async_copy(x_hbm.at[pl.ds(0, CHUNK)], rows_v, sem).wait()

    def start_dst(j, dst_v, sem):
        off = wid * EDGES_PER_WORKER + j * CHUNK
        pltpu.async_copy(dst_hbm.at[pl.ds(off, CHUNK)], dst_v, sem)

    def wait_dst(dst_v, sem):
        pltpu.make_async_copy(dst_hbm.at[pl.ds(0, CHUNK)], dst_v, sem).wait()

    HALF = CHUNK // 2

    def start_gather(j, rows_v, sem):
        pltpu.async_copy(x_hbm.at[src_v.at[j, pl.ds(0, HALF)]],
                         rows_v.at[pl.ds(0, HALF)], sem)
        pltpu.async_copy(x_hbm.at[src_v.at[j, pl.ds(HALF, HALF)]],
                         rows_v.at[pl.ds(HALF, HALF)], sem)

    def wait_gather2(rows_v, sem):
        pltpu.make_async_copy(x_hbm.at[pl.ds(0, HALF)],
                              rows_v.at[pl.ds(0, HALF)], sem).wait()
        pltpu.make_async_copy(x_hbm.at[pl.ds(0, HALF)],
                              rows_v.at[pl.ds(HALF, HALF)], sem).wait()

    start_dst(0, dst_a, sem_da)
    start_gather(0, rows_a, sem_a)

    def body(i, carry):
        j0 = 2 * i
        start_dst(j0 + 1, dst_b, sem_db)
        start_gather(j0 + 1, rows_b, sem_b)
        wait_gather2(rows_a, sem_a)
        wait_dst(dst_a, sem_da)
        pltpu.sync_copy(rows_a, agg_sh.at[dst_a], add=True)
        start_dst(j0 + 2, dst_a, sem_da)
        start_gather(j0 + 2, rows_a, sem_a)
        wait_gather2(rows_b, sem_b)
        wait_dst(dst_b, sem_db)
        pltpu.sync_copy(rows_b, agg_sh.at[dst_b], add=True)
        return carry

    lax.fori_loop(0, PAIRS, body, 0)
    wait_gather2(rows_a, sem_a)
    wait_dst(dst_a, sem_da)
    pltpu.sync_copy(rows_a, agg_sh.at[dst_a], add=True)

    plsc.subcore_barrier()
    pltpu.sync_copy(agg_sh.at[pl.ds(row0, ROWS_MAIN)],
                    out_hbm.at[pl.ds(cid * N_NODES + row0, ROWS_MAIN)])

    @pl.when(sid == NUM_SUBCORES - 1)
    def _copy_tail():
        pltpu.sync_copy(
            agg_sh.at[pl.ds(NUM_SUBCORES * ROWS_MAIN, ROWS_TAIL)],
            out_hbm.at[pl.ds(cid * N_NODES + NUM_SUBCORES * ROWS_MAIN,
                             ROWS_TAIL)])


def _mlp_body(eps_ref, x_ref, a0_ref, a1_ref, w1_ref, b1_ref, w2_ref, b2_ref,
              o_ref):
    h = x_ref[...] * eps_ref[0] + a0_ref[...] + a1_ref[...]
    h = jnp.dot(h, w1_ref[...], preferred_element_type=jnp.float32) + b1_ref[...]
    h = jnp.where(h > 0, h, 0.01 * h)
    h = jnp.dot(h, w2_ref[...], preferred_element_type=jnp.float32) + b2_ref[...]
    h = jnp.where(h > 0, h, 0.01 * h)
    o_ref[...] = h


_BLK = 1000

_mlp_call = pl.pallas_call(
    _mlp_body,
    out_shape=jax.ShapeDtypeStruct((N_NODES, D), jnp.float32),
    grid=(N_NODES // _BLK,),
    in_specs=[
        pl.BlockSpec(memory_space=pltpu.SMEM),
        pl.BlockSpec((_BLK, D), lambda i: (i, 0)),
        pl.BlockSpec((_BLK, D), lambda i: (i, 0)),
        pl.BlockSpec((_BLK, D), lambda i: (i, 0)),
        pl.BlockSpec((D, D), lambda i: (0, 0)),
        pl.BlockSpec((1, D), lambda i: (0, 0)),
        pl.BlockSpec((D, D), lambda i: (0, 0)),
        pl.BlockSpec((1, D), lambda i: (0, 0)),
    ],
    out_specs=pl.BlockSpec((_BLK, D), lambda i: (i, 0)),
)


def kernel(x, edge_index, eps, W1, b1, W2, b2):
    src = edge_index[0].reshape(NUM_WORKERS, CHUNKS_PER_WORKER, CHUNK)
    dst = edge_index[1]
    zeros = jnp.zeros((ROWS_MAIN, D), jnp.float32)
    agg = _sc_aggregate(x, src, dst, zeros)
    scale = jnp.reshape(1.0 + eps, (1,)).astype(jnp.float32)
    out = _mlp_call(scale, x, agg[:N_NODES], agg[N_NODES:],
                    W1.T, b1.reshape(1, D), W2.T, b2.reshape(1, D))
    return out


# speedup vs baseline: 3.1877x; 1.0426x over previous
import functools

import jax
import jax.numpy as jnp
from jax import lax
from jax.experimental import pallas as pl
from jax.experimental.pallas import tpu as pltpu
from jax.experimental.pallas import tpu_sc as plsc

N_NODES = 10000
N_EDGES = 320000
D = 128

NUM_CORES = 2
NUM_SUBCORES = 16
NUM_WORKERS = NUM_CORES * NUM_SUBCORES

CHUNK = 80
EDGES_PER_WORKER = N_EDGES // NUM_WORKERS
CHUNKS_PER_WORKER = EDGES_PER_WORKER // CHUNK
PAIRS = (CHUNKS_PER_WORKER - 1) // 2
ROWS_MAIN = 624
ROWS_TAIL = N_NODES - NUM_SUBCORES * ROWS_MAIN

_SC_MESH = plsc.VectorSubcoreMesh(core_axis_name="c", subcore_axis_name="s")


@functools.partial(
    pl.kernel,
    mesh=_SC_MESH,
    out_type=jax.ShapeDtypeStruct((NUM_CORES * N_NODES, D), jnp.float32),
    scratch_types=[
        pltpu.VMEM((CHUNKS_PER_WORKER, CHUNK), jnp.int32),
        pltpu.VMEM((CHUNK,), jnp.int32),
        pltpu.VMEM((CHUNK,), jnp.int32),
        pltpu.VMEM((CHUNK, D), jnp.float32),
        pltpu.VMEM((CHUNK, D), jnp.float32),
        pltpu.VMEM_SHARED((N_NODES, D), jnp.float32),
        pltpu.SemaphoreType.DMA,
        pltpu.SemaphoreType.DMA,
        pltpu.SemaphoreType.DMA,
        pltpu.SemaphoreType.DMA,
    ],
)
def _sc_aggregate(x_hbm, src_hbm, dst_hbm, zeros_hbm, out_hbm,
                  src_v, dst_a, dst_b, rows_a, rows_b, agg_sh,
                  sem_a, sem_b, sem_da, sem_db):
    cid = lax.axis_index("c")
    sid = lax.axis_index("s")
    wid = sid * NUM_CORES + cid

    row0 = sid * ROWS_MAIN
    pltpu.sync_copy(zeros_hbm, agg_sh.at[pl.ds(row0, ROWS_MAIN)])

    @pl.when(sid == NUM_SUBCORES - 1)
    def _zero_tail():
        pltpu.sync_copy(zeros_hbm.at[pl.ds(0, ROWS_TAIL)],
                        agg_sh.at[pl.ds(NUM_SUBCORES * ROWS_MAIN, ROWS_TAIL)])

    pltpu.sync_copy(src_hbm.at[wid], src_v)
    plsc.subcore_barrier()

    def wait_gather(rows_v, sem):
        pltpu.make_async_copy(x_hbm.at[pl.ds(0, CHUNK)], rows_v, sem).wait()

    def start_dst(j, dst_v, sem):
        off = wid * EDGES_PER_WORKER + j * CHUNK
        pltpu.async_copy(dst_hbm.at[pl.ds(off, CHUNK)], dst_v, sem)

    def wait_dst(dst_v, sem):
        pltpu.make_async_copy(dst_hbm.at[pl.ds(0, CHUNK)], dst_v, sem).wait()

    HALF = CHUNK // 2

    def start_gather(j, rows_v, sem):
        pltpu.async_copy(x_hbm.at[src_v.at[j, pl.ds(0, HALF)]],
                         rows_v.at[pl.ds(0, HALF)], sem)
        pltpu.async_copy(x_hbm.at[src_v.at[j, pl.ds(HALF, HALF)]],
                         rows_v.at[pl.ds(HALF, HALF)], sem)

    def wait_gather2(rows_v, sem):
        pltpu.make_async_copy(x_hbm.at[pl.ds(0, HALF)],
                              rows_v.at[pl.ds(0, HALF)], sem).wait()
        pltpu.make_async_copy(x_hbm.at[pl.ds(0, HALF)],
                              rows_v.at[pl.ds(HALF, HALF)], sem).wait()

    start_dst(0, dst_a, sem_da)
    start_gather(0, rows_a, sem_a)

    def body(i, carry):
        j0 = 2 * i
        start_dst(j0 + 1, dst_b, sem_db)
        start_gather(j0 + 1, rows_b, sem_b)
        wait_gather2(rows_a, sem_a)
        wait_dst(dst_a, sem_da)
        pltpu.sync_copy(rows_a, agg_sh.at[dst_a], add=True)
        start_dst(j0 + 2, dst_a, sem_da)
        start_gather(j0 + 2, rows_a, sem_a)
        wait_gather2(rows_b, sem_b)
        wait_dst(dst_b, sem_db)
        pltpu.sync_copy(rows_b, agg_sh.at[dst_b], add=True)
        return carry

    lax.fori_loop(0, PAIRS, body, 0)
    wait_gather2(rows_a, sem_a)
    wait_dst(dst_a, sem_da)
    pltpu.sync_copy(rows_a, agg_sh.at[dst_a], add=True)

    plsc.subcore_barrier()
    pltpu.sync_copy(agg_sh.at[pl.ds(row0, ROWS_MAIN)],
                    out_hbm.at[pl.ds(cid * N_NODES + row0, ROWS_MAIN)])

    @pl.when(sid == NUM_SUBCORES - 1)
    def _copy_tail():
        pltpu.sync_copy(
            agg_sh.at[pl.ds(NUM_SUBCORES * ROWS_MAIN, ROWS_TAIL)],
            out_hbm.at[pl.ds(cid * N_NODES + NUM_SUBCORES * ROWS_MAIN,
                             ROWS_TAIL)])


def _mlp_body(eps_ref, x_ref, a0_ref, a1_ref, w1_ref, b1_ref, w2_ref, b2_ref,
              o_ref):
    h = x_ref[...] * eps_ref[0] + a0_ref[...] + a1_ref[...]
    h = jnp.dot(h, w1_ref[...], preferred_element_type=jnp.float32) + b1_ref[...]
    h = jnp.where(h > 0, h, 0.01 * h)
    h = jnp.dot(h, w2_ref[...], preferred_element_type=jnp.float32) + b2_ref[...]
    h = jnp.where(h > 0, h, 0.01 * h)
    o_ref[...] = h


_BLK = 1000

_mlp_call = pl.pallas_call(
    _mlp_body,
    out_shape=jax.ShapeDtypeStruct((N_NODES, D), jnp.float32),
    grid=(N_NODES // _BLK,),
    in_specs=[
        pl.BlockSpec(memory_space=pltpu.SMEM),
        pl.BlockSpec((_BLK, D), lambda i: (i, 0)),
        pl.BlockSpec((_BLK, D), lambda i: (i, 0)),
        pl.BlockSpec((_BLK, D), lambda i: (i + N_NODES // _BLK, 0)),
        pl.BlockSpec((D, D), lambda i: (0, 0)),
        pl.BlockSpec((1, D), lambda i: (0, 0)),
        pl.BlockSpec((D, D), lambda i: (0, 0)),
        pl.BlockSpec((1, D), lambda i: (0, 0)),
    ],
    out_specs=pl.BlockSpec((_BLK, D), lambda i: (i, 0)),
)


def kernel(x, edge_index, eps, W1, b1, W2, b2):
    src = edge_index[0].reshape(NUM_WORKERS, CHUNKS_PER_WORKER, CHUNK)
    dst = edge_index[1]
    zeros = jnp.zeros((ROWS_MAIN, D), jnp.float32)
    agg = _sc_aggregate(x, src, dst, zeros)
    scale = jnp.reshape(1.0 + eps, (1,)).astype(jnp.float32)
    out = _mlp_call(scale, x, agg, agg,
                    W1.T, b1.reshape(1, D), W2.T, b2.reshape(1, D))
    return out
